# baseline (speedup 1.0000x reference)
import dataclasses
import functools
import math

import jax
import jax.numpy as jnp
from jax import lax
from jax.experimental import pallas as pl
from jax.experimental.pallas import tpu as pltpu
from jax.experimental.pallas import tpu_sc as plsc

DK = 64
NR = 2
BL = 64
SL = 2048
B = 32
NB = SL // BL
BR = B * NR
NBK = NB

_NEG_BIG = -1000000000.0
_NEG_SELF = -100000.0


def _sc_compiler_params():
    cp = pltpu.CompilerParams()
    if "needs_layout_passes" in pltpu.CompilerParams.__dataclass_fields__:
        cp = dataclasses.replace(cp, needs_layout_passes=False)
    return cp


NG = 16
GL = SL // NG


def _pairs_to_rows(x2):
    c = x2.shape[1] // 2
    a = x2[:, :c].reshape(-1, 1, c)
    b = x2[:, c:].reshape(-1, 1, c)
    return jnp.concatenate([a, b], axis=1).reshape(-1, c)


def _rows_to_pairs(x):
    c = x.shape[1]
    x3 = x.reshape(-1, 2, c)
    return jnp.concatenate([x3[:, 0, :], x3[:, 1, :]], axis=1)


def _hash_perm_body(q_ref, v_ref, rm_ref, qv_ref, p_ref):
    q = _pairs_to_rows(q_ref[0])
    v = _pairs_to_rows(v_ref[0])
    rm = rm_ref[0]
    qv_ref[0] = jnp.concatenate([q, v], axis=1)
    proj = jnp.dot(q, rm, preferred_element_type=jnp.float32)
    lane32f = lax.broadcasted_iota(jnp.int32, (SL, NBK), 1).astype(jnp.float32)
    ohs = []
    for r in range(NR):
        pr = proj[:, r * 16:(r + 1) * 16]
        logits = jnp.concatenate([pr, -pr], axis=1)
        m = jnp.max(logits, axis=1, keepdims=True)
        cand = jnp.where(logits == m, lane32f, 64.0)
        bucketf = jnp.min(cand, axis=1, keepdims=True)
        ohs.append((lane32f == bucketf).astype(jnp.float32))
    oh_f = jnp.concatenate(ohs, axis=1)
    oh_b = oh_f.astype(jnp.bfloat16).reshape(NG, GL, 2 * NBK)
    rg = lax.broadcasted_iota(jnp.int32, (GL, GL), 0)
    cg = lax.broadcasted_iota(jnp.int32, (GL, GL), 1)
    tri_g = jnp.broadcast_to((cg < rg).astype(jnp.bfloat16)[None],
                             (NG, GL, GL))
    local = lax.dot_general(tri_g, oh_b, (((2,), (1,)), ((0,), (0,))),
                            preferred_element_type=jnp.float32)
    gsum = jnp.sum(oh_b.astype(jnp.float32), axis=1)
    r16 = lax.broadcasted_iota(jnp.int32, (NG, NG), 0)
    c16 = lax.broadcasted_iota(jnp.int32, (NG, NG), 1)
    tri_16 = (c16 < r16).astype(jnp.bfloat16)
    prefix = jnp.dot(tri_16, gsum.astype(jnp.bfloat16),
                     preferred_element_type=jnp.float32)
    rank_mat = (local + prefix[:, None, :]).reshape(SL, 2 * NBK)
    hist_row = (prefix[NG - 1:NG, :] + gsum[NG - 1:NG, :])
    r64 = lax.broadcasted_iota(jnp.int32, (2 * NBK, 2 * NBK), 0)
    c64 = lax.broadcasted_iota(jnp.int32, (2 * NBK, 2 * NBK), 1)
    tri_u = ((r64 < c64) & ((r64 < NBK) == (c64 < NBK))).astype(jnp.float32)
    start_row = lax.dot(hist_row, tri_u,
                        precision=lax.Precision.HIGHEST,
                        preferred_element_type=jnp.float32)
    tmp = (rank_mat + start_row) * oh_f
    c2 = lax.broadcasted_iota(jnp.int32, (2 * NBK, NR), 0)
    r2 = lax.broadcasted_iota(jnp.int32, (2 * NBK, NR), 1)
    blockind = ((c2 // NBK) == r2).astype(jnp.float32)
    p2 = lax.dot(tmp, blockind, precision=lax.Precision.HIGHEST,
                 preferred_element_type=jnp.float32)
    p_ref[0] = p2.astype(jnp.int32).T


def _hash_perm(query, value, rm):
    q2 = query.reshape(B, SL // 2, 2 * DK)
    v2 = value.reshape(B, SL // 2, 2 * DK)
    return pl.pallas_call(
        _hash_perm_body,
        grid=(B,),
        in_specs=[
            pl.BlockSpec((1, SL // 2, 2 * DK), lambda b: (b, 0, 0)),
            pl.BlockSpec((1, SL // 2, 2 * DK), lambda b: (b, 0, 0)),
            pl.BlockSpec((1, DK, NR * 16), lambda b: (b, 0, 0)),
        ],
        out_specs=[
            pl.BlockSpec((1, SL, 2 * DK), lambda b: (b, 0, 0)),
            pl.BlockSpec((1, NR, SL), lambda b: (b, 0, 0)),
        ],
        out_shape=[
            jax.ShapeDtypeStruct((B, SL, 2 * DK), jnp.float32),
            jax.ShapeDtypeStruct((B, NR, SL), jnp.int32),
        ],
    )(q2, v2, rm)


def _sc_sort_gather_body(qv_hbm, p_hbm, qvs_hbm, sort_hbm,
                         pbuf, sbuf, sgbuf, rowbuf, sem):
    wid = lax.axis_index("s") * 2 + lax.axis_index("c")
    for r in range(NR):
        br = wid * NR + r
        b_off = wid * SL
        pltpu.sync_copy(p_hbm.at[wid, r], pbuf)

        @pl.loop(0, SL, step=16)
        def _(i):
            vals = lax.broadcasted_iota(jnp.int32, (16,), 0) + i
            idx = pbuf[pl.ds(i, 16)]
            plsc.store_scatter(sbuf, [idx], vals)
            plsc.store_scatter(sgbuf, [idx >> 7, idx & 127], vals + b_off)

        pltpu.sync_copy(sbuf, sort_hbm.at[br])
        for quarter in range(4):
            cps = [
                pltpu.async_copy(
                    qv_hbm.at[sgbuf.at[quarter * 4 + c]],
                    rowbuf.at[pl.ds(c * 128, 128)], sem)
                for c in range(4)
            ]
            for cp in cps:
                cp.wait()
            pltpu.sync_copy(rowbuf,
                            qvs_hbm.at[br, pl.ds(quarter * 512, 512)])


def _sc_sort_gather(qv, p):
    mesh = plsc.VectorSubcoreMesh(core_axis_name="c", subcore_axis_name="s")
    f = pl.kernel(
        _sc_sort_gather_body,
        out_type=(
            jax.ShapeDtypeStruct((BR, SL, 2 * DK), jnp.float32),
            jax.ShapeDtypeStruct((BR, SL), jnp.int32),
        ),
        mesh=mesh,
        scratch_types=[
            pltpu.VMEM((SL,), jnp.int32),
            pltpu.VMEM((SL,), jnp.int32),
            pltpu.VMEM((16, 128), jnp.int32),
            pltpu.VMEM((512, 2 * DK), jnp.float32),
            pltpu.SemaphoreType.DMA,
        ],
        compiler_params=_sc_compiler_params(),
    )
    return f(qv, p)


def _attn_body(qvs_ref, srow_ref, prow_ref, out_ref):
    qv = qvs_ref[0]
    q3 = qv[:, :DK].reshape(NB, BL, DK)
    v3 = qv[:, DK:].reshape(NB, BL, DK)
    srow = srow_ref[0]
    scol = srow.reshape(NB, BL, 1)
    pcol = prow_ref[0].reshape(NB, BL, 1)

    def chunkcat(x):
        prev = jnp.concatenate([x[NB - 1:NB], x[:NB - 1]], axis=0)
        return jnp.concatenate([prev, x], axis=1)

    k3 = chunkcat(q3)
    vc = chunkcat(v3)
    norm = jnp.sum(k3 * k3, axis=2, keepdims=True)
    k3 = k3 * lax.rsqrt(jnp.maximum(norm, 1e-12))
    qk = lax.dot_general(q3.astype(jnp.bfloat16), k3.astype(jnp.bfloat16),
                         (((2,), (2,)), ((0,), (0,))),
                         preferred_element_type=jnp.float32)
    qk = qk * (1.0 / math.sqrt(DK))

    krow = jnp.concatenate(
        [jnp.concatenate([srow[NB - 1:NB], srow[:NB - 1]], axis=0), srow],
        axis=1).reshape(NB, 1, 2 * BL)
    qcol = scol
    qk = jnp.where(qcol < krow, _NEG_BIG, qk)
    qk = jnp.where(qcol == krow, _NEG_SELF, qk)

    kcol_part = jnp.concatenate(
        [jnp.concatenate([pcol[NB - 1:NB], pcol[:NB - 1]], axis=0), pcol],
        axis=1)
    eq = (kcol_part == krow).astype(jnp.float32)
    count = 1.0 + jnp.sum(eq, axis=1).reshape(NB, 1, 2 * BL)

    m = jnp.max(qk, axis=2, keepdims=True)
    e = jnp.exp(qk - m)
    s = jnp.sum(e, axis=2, keepdims=True)
    lse = jnp.log(s) + m
    sm = e / (s * count)
    out = lax.dot_general(sm.astype(jnp.bfloat16), vc.astype(jnp.bfloat16),
                          (((2,), (1,)), ((0,), (0,))),
                          preferred_element_type=jnp.float32)
    pack = jnp.concatenate(
        [out, jnp.broadcast_to(lse, (NB, BL, DK))], axis=2)
    out_ref[0] = pack.reshape(SL, 2 * DK)


def _attention(qvs, sort):
    s3 = sort.reshape(BR, NB, BL)
    return pl.pallas_call(
        _attn_body,
        grid=(BR,),
        in_specs=[
            pl.BlockSpec((1, SL, 2 * DK), lambda i: (i, 0, 0)),
            pl.BlockSpec((1, NB, BL), lambda i: (i, 0, 0)),
            pl.BlockSpec((1, NB, BL), lambda i: (i + 1 - 2 * (i % 2), 0, 0)),
        ],
        out_specs=pl.BlockSpec((1, SL, 2 * DK), lambda i: (i, 0, 0)),
        out_shape=jax.ShapeDtypeStruct((BR, SL, 2 * DK), jnp.float32),
    )(qvs, s3, s3)


def _sc_unsort_body(of_hbm, p_hbm, og_hbm, pbuf, pgbuf, rowbuf, sem):
    wid = lax.axis_index("s") * 2 + lax.axis_index("c")
    for r in range(NR):
        br = wid * NR + r
        row_off = br * SL
        pltpu.sync_copy(p_hbm.at[wid, r], pbuf)

        @pl.loop(0, SL, step=16)
        def _(i):
            idx = pbuf[pl.ds(i, 16)]
            pgbuf[i >> 7, pl.ds(i & 127, 16)] = idx + row_off

        for quarter in range(4):
            cps = [
                pltpu.async_copy(
                    of_hbm.at[pgbuf.at[quarter * 4 + c]],
                    rowbuf.at[pl.ds(c * 128, 128)], sem)
                for c in range(4)
            ]
            for cp in cps:
                cp.wait()
            pltpu.sync_copy(rowbuf, og_hbm.at[br, pl.ds(quarter * 512, 512)])


def _sc_unsort(outflat, p):
    mesh = plsc.VectorSubcoreMesh(core_axis_name="c", subcore_axis_name="s")
    f = pl.kernel(
        _sc_unsort_body,
        out_type=jax.ShapeDtypeStruct((BR, SL, 2 * DK), jnp.float32),
        mesh=mesh,
        scratch_types=[
            pltpu.VMEM((SL,), jnp.int32),
            pltpu.VMEM((16, 128), jnp.int32),
            pltpu.VMEM((512, 2 * DK), jnp.float32),
            pltpu.SemaphoreType.DMA,
        ],
        compiler_params=_sc_compiler_params(),
    )
    return f(outflat, p)


def _combine_body(p0_ref, p1_ref, out_ref):
    p0 = p0_ref[0]
    p1 = p1_ref[0]
    o0, l0 = p0[:, :DK], p0[:, DK:DK + 1]
    o1, l1 = p1[:, :DK], p1[:, DK:DK + 1]
    m = jnp.maximum(l0, l1)
    e0 = jnp.exp(l0 - m)
    e1 = jnp.exp(l1 - m)
    den = e0 + e1
    res = (e0 / den) * o0 + (e1 / den) * o1
    out_ref[0] = _rows_to_pairs(res)


def _combine(outg):
    return pl.pallas_call(
        _combine_body,
        grid=(B,),
        in_specs=[
            pl.BlockSpec((1, SL, 2 * DK), lambda b: (2 * b, 0, 0)),
            pl.BlockSpec((1, SL, 2 * DK), lambda b: (2 * b + 1, 0, 0)),
        ],
        out_specs=pl.BlockSpec((1, SL // 2, 2 * DK), lambda b: (b, 0, 0)),
        out_shape=jax.ShapeDtypeStruct((B, SL // 2, 2 * DK), jnp.float32),
    )(outg, outg)


def kernel(query, value, rand_matrix):
    rm = rand_matrix.reshape(B, DK, NR * 16)
    qv, p = _hash_perm(query, value, rm)
    qvs, sort = _sc_sort_gather(qv.reshape(B * SL, 2 * DK), p)
    out_s = _attention(qvs, sort)
    outg = _sc_unsort(out_s.reshape(BR * SL, 2 * DK), p)
    return _combine(outg).reshape(B, SL, DK)


# speedup vs baseline: 1.2102x; 1.2102x over previous
import dataclasses
import functools
import math

import jax
import jax.numpy as jnp
from jax import lax
from jax.experimental import pallas as pl
from jax.experimental.pallas import tpu as pltpu
from jax.experimental.pallas import tpu_sc as plsc

DK = 64
NR = 2
BL = 64
SL = 2048
B = 32
NB = SL // BL
BR = B * NR
NBK = NB

_NEG_BIG = -1000000000.0
_NEG_SELF = -100000.0


def _sc_compiler_params():
    cp = pltpu.CompilerParams()
    if "needs_layout_passes" in pltpu.CompilerParams.__dataclass_fields__:
        cp = dataclasses.replace(cp, needs_layout_passes=False)
    return cp


NG = 16
GL = SL // NG


def _pairs_to_rows(x2):
    c = x2.shape[1] // 2
    a = x2[:, :c].reshape(-1, 1, c)
    b = x2[:, c:].reshape(-1, 1, c)
    return jnp.concatenate([a, b], axis=1).reshape(-1, c)


def _rows_to_pairs(x):
    c = x.shape[1]
    x3 = x.reshape(-1, 2, c)
    return jnp.concatenate([x3[:, 0, :], x3[:, 1, :]], axis=1)


def _hash_perm_body(q_ref, v_ref, rm_ref, qv_ref, p_ref):
    q = q_ref[0]
    v = v_ref[0]
    rm = rm_ref[0]
    qv_ref[0] = jnp.concatenate([q, v], axis=1)
    proj = jnp.dot(q, rm, preferred_element_type=jnp.float32)
    lane32f = lax.broadcasted_iota(jnp.int32, (SL, NBK), 1).astype(jnp.float32)
    ohs = []
    for r in range(NR):
        pr = proj[:, r * 16:(r + 1) * 16]
        logits = jnp.concatenate([pr, -pr], axis=1)
        m = jnp.max(logits, axis=1, keepdims=True)
        cand = jnp.where(logits == m, lane32f, 64.0)
        bucketf = jnp.min(cand, axis=1, keepdims=True)
        ohs.append((lane32f == bucketf).astype(jnp.float32))
    oh_f = jnp.concatenate(ohs, axis=1)
    oh_b = oh_f.astype(jnp.bfloat16).reshape(NG, GL, 2 * NBK)
    rg = lax.broadcasted_iota(jnp.int32, (GL, GL), 0)
    cg = lax.broadcasted_iota(jnp.int32, (GL, GL), 1)
    tri_g = jnp.broadcast_to((cg < rg).astype(jnp.bfloat16)[None],
                             (NG, GL, GL))
    local = lax.dot_general(tri_g, oh_b, (((2,), (1,)), ((0,), (0,))),
                            preferred_element_type=jnp.float32)
    gsum = jnp.sum(oh_b.astype(jnp.float32), axis=1)
    r16 = lax.broadcasted_iota(jnp.int32, (NG, NG), 0)
    c16 = lax.broadcasted_iota(jnp.int32, (NG, NG), 1)
    tri_16 = (c16 < r16).astype(jnp.bfloat16)
    prefix = jnp.dot(tri_16, gsum.astype(jnp.bfloat16),
                     preferred_element_type=jnp.float32)
    rank_mat = (local + prefix[:, None, :]).reshape(SL, 2 * NBK)
    hist_row = (prefix[NG - 1:NG, :] + gsum[NG - 1:NG, :])
    r64 = lax.broadcasted_iota(jnp.int32, (2 * NBK, 2 * NBK), 0)
    c64 = lax.broadcasted_iota(jnp.int32, (2 * NBK, 2 * NBK), 1)
    tri_u = ((r64 < c64) & ((r64 < NBK) == (c64 < NBK))).astype(jnp.float32)
    start_row = lax.dot(hist_row, tri_u,
                        precision=lax.Precision.HIGHEST,
                        preferred_element_type=jnp.float32)
    tmp = (rank_mat + start_row) * oh_f
    c2 = lax.broadcasted_iota(jnp.int32, (2 * NBK, NR), 0)
    r2 = lax.broadcasted_iota(jnp.int32, (2 * NBK, NR), 1)
    blockind = ((c2 // NBK) == r2).astype(jnp.float32)
    p2 = lax.dot(tmp, blockind, precision=lax.Precision.HIGHEST,
                 preferred_element_type=jnp.float32)
    p_ref[0] = p2.astype(jnp.int32).T


def _hash_perm(query, value, rm):
    return pl.pallas_call(
        _hash_perm_body,
        grid=(B,),
        in_specs=[
            pl.BlockSpec((1, SL, DK), lambda b: (b, 0, 0)),
            pl.BlockSpec((1, SL, DK), lambda b: (b, 0, 0)),
            pl.BlockSpec((1, DK, NR * 16), lambda b: (b, 0, 0)),
        ],
        out_specs=[
            pl.BlockSpec((1, SL, 2 * DK), lambda b: (b, 0, 0)),
            pl.BlockSpec((1, NR, SL), lambda b: (b, 0, 0)),
        ],
        out_shape=[
            jax.ShapeDtypeStruct((B, SL, 2 * DK), jnp.float32),
            jax.ShapeDtypeStruct((B, NR, SL), jnp.int32),
        ],
    )(query, value, rm)


def _sc_sort_gather_body(qv_hbm, p_hbm, qvs_hbm, sort_hbm,
                         pbuf, sbuf, sgbuf, rowbuf, sem):
    wid = lax.axis_index("s") * 2 + lax.axis_index("c")
    for r in range(NR):
        br = wid * NR + r
        b_off = wid * SL
        pltpu.sync_copy(p_hbm.at[wid, r], pbuf)

        @pl.loop(0, SL, step=16)
        def _(i):
            vals = lax.broadcasted_iota(jnp.int32, (16,), 0) + i
            idx = pbuf[pl.ds(i, 16)]
            plsc.store_scatter(sbuf, [idx], vals)
            plsc.store_scatter(sgbuf, [idx >> 7, idx & 127], vals + b_off)

        pltpu.sync_copy(sbuf, sort_hbm.at[br])
        for quarter in range(4):
            cps = [
                pltpu.async_copy(
                    qv_hbm.at[sgbuf.at[quarter * 4 + c]],
                    rowbuf.at[pl.ds(c * 128, 128)], sem)
                for c in range(4)
            ]
            for cp in cps:
                cp.wait()
            pltpu.sync_copy(rowbuf,
                            qvs_hbm.at[br, pl.ds(quarter * 512, 512)])


def _sc_sort_gather(qv, p):
    mesh = plsc.VectorSubcoreMesh(core_axis_name="c", subcore_axis_name="s")
    f = pl.kernel(
        _sc_sort_gather_body,
        out_type=(
            jax.ShapeDtypeStruct((BR, SL, 2 * DK), jnp.float32),
            jax.ShapeDtypeStruct((BR, SL), jnp.int32),
        ),
        mesh=mesh,
        scratch_types=[
            pltpu.VMEM((SL,), jnp.int32),
            pltpu.VMEM((SL,), jnp.int32),
            pltpu.VMEM((16, 128), jnp.int32),
            pltpu.VMEM((512, 2 * DK), jnp.float32),
            pltpu.SemaphoreType.DMA,
        ],
        compiler_params=_sc_compiler_params(),
    )
    return f(qv, p)


def _attn_body(qvs_ref, srow_ref, prow_ref, out_ref):
    qv = qvs_ref[0]
    q3 = qv[:, :DK].reshape(NB, BL, DK)
    v3 = qv[:, DK:].reshape(NB, BL, DK)
    srow = srow_ref[0]
    scol = srow.reshape(NB, BL, 1)
    pcol = prow_ref[0].reshape(NB, BL, 1)

    def chunkcat(x):
        prev = jnp.concatenate([x[NB - 1:NB], x[:NB - 1]], axis=0)
        return jnp.concatenate([prev, x], axis=1)

    k3 = chunkcat(q3)
    vc = chunkcat(v3)
    norm = jnp.sum(k3 * k3, axis=2, keepdims=True)
    k3 = k3 * lax.rsqrt(jnp.maximum(norm, 1e-12))
    qk = lax.dot_general(q3, k3, (((2,), (2,)), ((0,), (0,))),
                         preferred_element_type=jnp.float32)
    qk = qk * (1.0 / math.sqrt(DK))

    krow = jnp.concatenate(
        [jnp.concatenate([srow[NB - 1:NB], srow[:NB - 1]], axis=0), srow],
        axis=1).reshape(NB, 1, 2 * BL)
    qcol = scol
    qk = jnp.where(qcol < krow, _NEG_BIG, qk)
    qk = jnp.where(qcol == krow, _NEG_SELF, qk)

    kcol_part = jnp.concatenate(
        [jnp.concatenate([pcol[NB - 1:NB], pcol[:NB - 1]], axis=0), pcol],
        axis=1)
    eq = (kcol_part == krow).astype(jnp.float32)
    count = 1.0 + jnp.sum(eq, axis=1).reshape(NB, 1, 2 * BL)

    m = jnp.max(qk, axis=2, keepdims=True)
    e = jnp.exp(qk - m)
    s = jnp.sum(e, axis=2, keepdims=True)
    lse = jnp.log(s) + m
    sm = e / (s * count)
    out = lax.dot_general(sm, vc, (((2,), (1,)), ((0,), (0,))),
                          preferred_element_type=jnp.float32)
    pack = jnp.concatenate(
        [out, jnp.broadcast_to(lse, (NB, BL, DK))], axis=2)
    out_ref[0] = pack.reshape(SL, 2 * DK)


def _attention(qvs, sort):
    s3 = sort.reshape(BR, NB, BL)
    return pl.pallas_call(
        _attn_body,
        grid=(BR,),
        in_specs=[
            pl.BlockSpec((1, SL, 2 * DK), lambda i: (i, 0, 0)),
            pl.BlockSpec((1, NB, BL), lambda i: (i, 0, 0)),
            pl.BlockSpec((1, NB, BL), lambda i: (i + 1 - 2 * (i % 2), 0, 0)),
        ],
        out_specs=pl.BlockSpec((1, SL, 2 * DK), lambda i: (i, 0, 0)),
        out_shape=jax.ShapeDtypeStruct((BR, SL, 2 * DK), jnp.float32),
    )(qvs, s3, s3)


def _sc_unsort_body(of_hbm, p_hbm, og_hbm, pbuf, pgbuf, rowbuf, sem):
    wid = lax.axis_index("s") * 2 + lax.axis_index("c")
    for r in range(NR):
        br = wid * NR + r
        row_off = br * SL
        pltpu.sync_copy(p_hbm.at[wid, r], pbuf)

        @pl.loop(0, SL, step=16)
        def _(i):
            idx = pbuf[pl.ds(i, 16)]
            pgbuf[i >> 7, pl.ds(i & 127, 16)] = idx + row_off

        for quarter in range(4):
            cps = [
                pltpu.async_copy(
                    of_hbm.at[pgbuf.at[quarter * 4 + c]],
                    rowbuf.at[pl.ds(c * 128, 128)], sem)
                for c in range(4)
            ]
            for cp in cps:
                cp.wait()
            pltpu.sync_copy(rowbuf, og_hbm.at[br, pl.ds(quarter * 512, 512)])


def _sc_unsort(outflat, p):
    mesh = plsc.VectorSubcoreMesh(core_axis_name="c", subcore_axis_name="s")
    f = pl.kernel(
        _sc_unsort_body,
        out_type=jax.ShapeDtypeStruct((BR, SL, 2 * DK), jnp.float32),
        mesh=mesh,
        scratch_types=[
            pltpu.VMEM((SL,), jnp.int32),
            pltpu.VMEM((16, 128), jnp.int32),
            pltpu.VMEM((512, 2 * DK), jnp.float32),
            pltpu.SemaphoreType.DMA,
        ],
        compiler_params=_sc_compiler_params(),
    )
    return f(outflat, p)


def _combine_body(p0_ref, p1_ref, out_ref):
    p0 = p0_ref[0]
    p1 = p1_ref[0]
    o0, l0 = p0[:, :DK], p0[:, DK:DK + 1]
    o1, l1 = p1[:, :DK], p1[:, DK:DK + 1]
    m = jnp.maximum(l0, l1)
    e0 = jnp.exp(l0 - m)
    e1 = jnp.exp(l1 - m)
    den = e0 + e1
    res = (e0 / den) * o0 + (e1 / den) * o1
    out_ref[0] = _rows_to_pairs(res)


def _combine(outg):
    return pl.pallas_call(
        _combine_body,
        grid=(B,),
        in_specs=[
            pl.BlockSpec((1, SL, 2 * DK), lambda b: (2 * b, 0, 0)),
            pl.BlockSpec((1, SL, 2 * DK), lambda b: (2 * b + 1, 0, 0)),
        ],
        out_specs=pl.BlockSpec((1, SL // 2, 2 * DK), lambda b: (b, 0, 0)),
        out_shape=jax.ShapeDtypeStruct((B, SL // 2, 2 * DK), jnp.float32),
    )(outg, outg)


def kernel(query, value, rand_matrix):
    rm = rand_matrix.reshape(B, DK, NR * 16)
    qv, p = _hash_perm(query, value, rm)
    qvs, sort = _sc_sort_gather(qv.reshape(B * SL, 2 * DK), p)
    out_s = _attention(qvs, sort)
    outg = _sc_unsort(out_s.reshape(BR * SL, 2 * DK), p)
    return _combine(outg).reshape(B, SL, DK)


# speedup vs baseline: 1.2788x; 1.0566x over previous
import dataclasses
import functools
import math

import jax
import jax.numpy as jnp
from jax import lax
from jax.experimental import pallas as pl
from jax.experimental.pallas import tpu as pltpu
from jax.experimental.pallas import tpu_sc as plsc

DK = 64
NR = 2
BL = 64
SL = 2048
B = 32
NB = SL // BL
BR = B * NR
NBK = NB

_NEG_BIG = -1000000000.0
_NEG_SELF = -100000.0


def _sc_compiler_params():
    cp = pltpu.CompilerParams()
    if "needs_layout_passes" in pltpu.CompilerParams.__dataclass_fields__:
        cp = dataclasses.replace(cp, needs_layout_passes=False)
    return cp


NG = 16
GL = SL // NG


def _pairs_to_rows(x2):
    c = x2.shape[1] // 2
    a = x2[:, :c].reshape(-1, 1, c)
    b = x2[:, c:].reshape(-1, 1, c)
    return jnp.concatenate([a, b], axis=1).reshape(-1, c)


def _rows_to_pairs(x):
    c = x.shape[1]
    x3 = x.reshape(-1, 2, c)
    return jnp.concatenate([x3[:, 0, :], x3[:, 1, :]], axis=1)


def _hash_perm_body(q_ref, v_ref, rm_ref, qv_ref, p_ref):
    q = q_ref[0]
    v = v_ref[0]
    rm = rm_ref[0]
    qv_ref[0] = jnp.concatenate([q, v], axis=1)
    proj = jnp.dot(q, rm, preferred_element_type=jnp.float32)
    lane32f = lax.broadcasted_iota(jnp.int32, (SL, NBK), 1).astype(jnp.float32)
    ohs = []
    for r in range(NR):
        pr = proj[:, r * 16:(r + 1) * 16]
        logits = jnp.concatenate([pr, -pr], axis=1)
        m = jnp.max(logits, axis=1, keepdims=True)
        cand = jnp.where(logits == m, lane32f, 64.0)
        bucketf = jnp.min(cand, axis=1, keepdims=True)
        ohs.append((lane32f == bucketf).astype(jnp.float32))
    oh_f = jnp.concatenate(ohs, axis=1)
    oh_b = oh_f.astype(jnp.bfloat16).reshape(NG, GL, 2 * NBK)
    rg = lax.broadcasted_iota(jnp.int32, (GL, GL), 0)
    cg = lax.broadcasted_iota(jnp.int32, (GL, GL), 1)
    tri_g = jnp.broadcast_to((cg < rg).astype(jnp.bfloat16)[None],
                             (NG, GL, GL))
    local = lax.dot_general(tri_g, oh_b, (((2,), (1,)), ((0,), (0,))),
                            preferred_element_type=jnp.float32)
    gsum = jnp.sum(oh_b.astype(jnp.float32), axis=1)
    r16 = lax.broadcasted_iota(jnp.int32, (NG, NG), 0)
    c16 = lax.broadcasted_iota(jnp.int32, (NG, NG), 1)
    tri_16 = (c16 < r16).astype(jnp.bfloat16)
    prefix = jnp.dot(tri_16, gsum.astype(jnp.bfloat16),
                     preferred_element_type=jnp.float32)
    rank_mat = (local + prefix[:, None, :]).reshape(SL, 2 * NBK)
    hist_row = (prefix[NG - 1:NG, :] + gsum[NG - 1:NG, :])
    r64 = lax.broadcasted_iota(jnp.int32, (2 * NBK, 2 * NBK), 0)
    c64 = lax.broadcasted_iota(jnp.int32, (2 * NBK, 2 * NBK), 1)
    tri_u = ((r64 < c64) & ((r64 < NBK) == (c64 < NBK))).astype(jnp.float32)
    start_row = lax.dot(hist_row, tri_u,
                        precision=lax.Precision.HIGHEST,
                        preferred_element_type=jnp.float32)
    tmp = (rank_mat + start_row) * oh_f
    c2 = lax.broadcasted_iota(jnp.int32, (2 * NBK, NR), 0)
    r2 = lax.broadcasted_iota(jnp.int32, (2 * NBK, NR), 1)
    blockind = ((c2 // NBK) == r2).astype(jnp.float32)
    p2 = lax.dot(tmp, blockind, precision=lax.Precision.HIGHEST,
                 preferred_element_type=jnp.float32)
    p_ref[0] = p2.astype(jnp.int32).T


def _hash_perm(query, value, rm):
    nb = query.shape[0]
    return pl.pallas_call(
        _hash_perm_body,
        grid=(nb,),
        in_specs=[
            pl.BlockSpec((1, SL, DK), lambda b: (b, 0, 0)),
            pl.BlockSpec((1, SL, DK), lambda b: (b, 0, 0)),
            pl.BlockSpec((1, DK, NR * 16), lambda b: (b, 0, 0)),
        ],
        out_specs=[
            pl.BlockSpec((1, SL, 2 * DK), lambda b: (b, 0, 0)),
            pl.BlockSpec((1, NR, SL), lambda b: (b, 0, 0)),
        ],
        out_shape=[
            jax.ShapeDtypeStruct((nb, SL, 2 * DK), jnp.float32),
            jax.ShapeDtypeStruct((nb, NR, SL), jnp.int32),
        ],
    )(query, value, rm)


def _sc_sort_gather(qv, p):
    nb = p.shape[0]
    nt = nb * NR

    def body(qv_hbm, p_hbm, qvs_hbm, sort_hbm, pbuf, sbuf, sgbuf, rowbuf, sem):
        wid = lax.axis_index("s") * 2 + lax.axis_index("c")
        for t0 in range(0, nt, 32):
            t = t0 + wid
            b = t // NR
            r = t - b * NR
            b_off = b * SL
            pltpu.sync_copy(p_hbm.at[b, r], pbuf)

            @pl.loop(0, SL, step=16)
            def _(i):
                vals = lax.broadcasted_iota(jnp.int32, (16,), 0) + i
                idx = pbuf[pl.ds(i, 16)]
                plsc.store_scatter(sbuf, [idx], vals)
                plsc.store_scatter(sgbuf, [idx >> 7, idx & 127], vals + b_off)

            pltpu.sync_copy(sbuf, sort_hbm.at[t])
            for quarter in range(4):
                cps = [
                    pltpu.async_copy(
                        qv_hbm.at[sgbuf.at[quarter * 4 + c]],
                        rowbuf.at[pl.ds(c * 128, 128)], sem)
                    for c in range(4)
                ]
                for cp in cps:
                    cp.wait()
                pltpu.sync_copy(rowbuf,
                                qvs_hbm.at[t, pl.ds(quarter * 512, 512)])

    mesh = plsc.VectorSubcoreMesh(core_axis_name="c", subcore_axis_name="s")
    f = pl.kernel(
        body,
        out_type=(
            jax.ShapeDtypeStruct((nt, SL, 2 * DK), jnp.float32),
            jax.ShapeDtypeStruct((nt, SL), jnp.int32),
        ),
        mesh=mesh,
        scratch_types=[
            pltpu.VMEM((SL,), jnp.int32),
            pltpu.VMEM((SL,), jnp.int32),
            pltpu.VMEM((16, 128), jnp.int32),
            pltpu.VMEM((512, 2 * DK), jnp.float32),
            pltpu.SemaphoreType.DMA,
        ],
        compiler_params=_sc_compiler_params(),
    )
    return f(qv, p)


def _attn_body(qvs_ref, srow_ref, prow_ref, out_ref):
    qv = qvs_ref[0]
    q3 = qv[:, :DK].reshape(NB, BL, DK)
    v3 = qv[:, DK:].reshape(NB, BL, DK)
    srow = srow_ref[0]
    scol = srow.reshape(NB, BL, 1)
    pcol = prow_ref[0].reshape(NB, BL, 1)

    def chunkcat(x):
        prev = jnp.concatenate([x[NB - 1:NB], x[:NB - 1]], axis=0)
        return jnp.concatenate([prev, x], axis=1)

    k3 = chunkcat(q3)
    vc = chunkcat(v3)
    norm = jnp.sum(k3 * k3, axis=2, keepdims=True)
    k3 = k3 * lax.rsqrt(jnp.maximum(norm, 1e-12))
    qk = lax.dot_general(q3, k3, (((2,), (2,)), ((0,), (0,))),
                         preferred_element_type=jnp.float32)
    qk = qk * (1.0 / math.sqrt(DK))

    krow = jnp.concatenate(
        [jnp.concatenate([srow[NB - 1:NB], srow[:NB - 1]], axis=0), srow],
        axis=1).reshape(NB, 1, 2 * BL)
    qcol = scol
    qk = jnp.where(qcol < krow, _NEG_BIG, qk)
    qk = jnp.where(qcol == krow, _NEG_SELF, qk)

    kcol_part = jnp.concatenate(
        [jnp.concatenate([pcol[NB - 1:NB], pcol[:NB - 1]], axis=0), pcol],
        axis=1)
    eq = (kcol_part == krow).astype(jnp.float32)
    count = 1.0 + jnp.sum(eq, axis=1).reshape(NB, 1, 2 * BL)

    m = jnp.max(qk, axis=2, keepdims=True)
    e = jnp.exp(qk - m)
    s = jnp.sum(e, axis=2, keepdims=True)
    lse = jnp.log(s) + m
    sm = e / (s * count)
    out = lax.dot_general(sm, vc, (((2,), (1,)), ((0,), (0,))),
                          preferred_element_type=jnp.float32)
    pack = jnp.concatenate(
        [out, jnp.broadcast_to(lse, (NB, BL, DK))], axis=2)
    out_ref[0] = pack.reshape(SL, 2 * DK)


def _attention(qvs, sort):
    nt = qvs.shape[0]
    s3 = sort.reshape(nt, NB, BL)
    return pl.pallas_call(
        _attn_body,
        grid=(nt,),
        in_specs=[
            pl.BlockSpec((1, SL, 2 * DK), lambda i: (i, 0, 0)),
            pl.BlockSpec((1, NB, BL), lambda i: (i, 0, 0)),
            pl.BlockSpec((1, NB, BL), lambda i: (i + 1 - 2 * (i % 2), 0, 0)),
        ],
        out_specs=pl.BlockSpec((1, SL, 2 * DK), lambda i: (i, 0, 0)),
        out_shape=jax.ShapeDtypeStruct((nt, SL, 2 * DK), jnp.float32),
    )(qvs, s3, s3)


def _sc_unsort(outflat, p):
    nb = p.shape[0]
    nt = nb * NR

    def body(of_hbm, p_hbm, og_hbm, pbuf, pgbuf, rowbuf, sem):
        wid = lax.axis_index("s") * 2 + lax.axis_index("c")
        for t0 in range(0, nt, 32):
            t = t0 + wid
            b = t // NR
            r = t - b * NR
            row_off = t * SL
            pltpu.sync_copy(p_hbm.at[b, r], pbuf)

            @pl.loop(0, SL, step=16)
            def _(i):
                idx = pbuf[pl.ds(i, 16)]
                pgbuf[i >> 7, pl.ds(i & 127, 16)] = idx + row_off

            for quarter in range(4):
                cps = [
                    pltpu.async_copy(
                        of_hbm.at[pgbuf.at[quarter * 4 + c]],
                        rowbuf.at[pl.ds(c * 128, 128)], sem)
                    for c in range(4)
                ]
                for cp in cps:
                    cp.wait()
                pltpu.sync_copy(rowbuf,
                                og_hbm.at[t, pl.ds(quarter * 512, 512)])

    mesh = plsc.VectorSubcoreMesh(core_axis_name="c", subcore_axis_name="s")
    f = pl.kernel(
        body,
        out_type=jax.ShapeDtypeStruct((nt, SL, 2 * DK), jnp.float32),
        mesh=mesh,
        scratch_types=[
            pltpu.VMEM((SL,), jnp.int32),
            pltpu.VMEM((16, 128), jnp.int32),
            pltpu.VMEM((512, 2 * DK), jnp.float32),
            pltpu.SemaphoreType.DMA,
        ],
        compiler_params=_sc_compiler_params(),
    )
    return f(outflat, p)


def _combine_body(p0_ref, p1_ref, out_ref):
    p0 = p0_ref[0]
    p1 = p1_ref[0]
    o0, l0 = p0[:, :DK], p0[:, DK:DK + 1]
    o1, l1 = p1[:, :DK], p1[:, DK:DK + 1]
    m = jnp.maximum(l0, l1)
    e0 = jnp.exp(l0 - m)
    e1 = jnp.exp(l1 - m)
    den = e0 + e1
    res = (e0 / den) * o0 + (e1 / den) * o1
    out_ref[0] = _rows_to_pairs(res)


def _combine(outg):
    nb = outg.shape[0] // NR
    return pl.pallas_call(
        _combine_body,
        grid=(nb,),
        in_specs=[
            pl.BlockSpec((1, SL, 2 * DK), lambda b: (2 * b, 0, 0)),
            pl.BlockSpec((1, SL, 2 * DK), lambda b: (2 * b + 1, 0, 0)),
        ],
        out_specs=pl.BlockSpec((1, SL // 2, 2 * DK), lambda b: (b, 0, 0)),
        out_shape=jax.ShapeDtypeStruct((nb, SL // 2, 2 * DK), jnp.float32),
    )(outg, outg)


def _half_pipeline(q, v, rm):
    qv, p = _hash_perm(q, v, rm)
    qvs, sort = _sc_sort_gather(qv.reshape(-1, 2 * DK), p)
    out_s = _attention(qvs, sort)
    outg = _sc_unsort(out_s.reshape(-1, 2 * DK), p)
    return _combine(outg)


def kernel(query, value, rand_matrix):
    rm = rand_matrix.reshape(B, DK, NR * 16)
    h = B // 2
    out0 = _half_pipeline(query[:h], value[:h], rm[:h])
    out1 = _half_pipeline(query[h:], value[h:], rm[h:])
    return jnp.concatenate([out0, out1], axis=0).reshape(B, SL, DK)
